# baseline (speedup 1.0000x reference)
import functools

import jax
import jax.numpy as jnp
from jax import lax
from jax.experimental import pallas as pl
from jax.experimental.pallas import tpu as pltpu
from jax.experimental.pallas import tpu_sc as plsc

L = 16
NC = 2
NS = 16
NW = NC * NS
E = 64
CHUNK = 128


@functools.lru_cache(maxsize=None)
def _build(B):
  BW = B // NW
  NCH = BW // CHUNK
  mesh = plsc.VectorSubcoreMesh(core_axis_name="c", subcore_axis_name="s")

  @functools.partial(
      pl.kernel,
      out_type=jax.ShapeDtypeStruct((B,), jnp.float32),
      mesh=mesh,
      compiler_params=pltpu.CompilerParams(use_tc_tiling_on_sc=False),
      scratch_types=[
          pltpu.VMEM((NCH, CHUNK), jnp.int32),
          pltpu.VMEM((NCH, CHUNK), jnp.int32),
          pltpu.VMEM((BW, E), jnp.float32),
          pltpu.VMEM((BW, E), jnp.float32),
          pltpu.VMEM((2 * E,), jnp.float32),
          pltpu.VMEM((L,), jnp.float32),
          pltpu.VMEM((BW,), jnp.float32),
          pltpu.SemaphoreType.DMA,
          pltpu.SemaphoreType.DMA,
      ],
  )
  def k(user_hbm, power_hbm, ut_hbm, pt_hbm, fcw_hbm, fcb_hbm, out_hbm,
        uidx_v, pidx_v, urows_v, prows_v, w_v, b_v, out_v, usem, psem):
    wid = lax.axis_index("s") * NC + lax.axis_index("c")
    base = wid * BW

    for j in range(NCH):
      pltpu.sync_copy(user_hbm.at[pl.ds(base + j * CHUNK, CHUNK)],
                      uidx_v.at[j])
      pltpu.sync_copy(power_hbm.at[pl.ds(base + j * CHUNK, CHUNK)],
                      pidx_v.at[j])
    pltpu.sync_copy(fcw_hbm.at[0], w_v)
    pltpu.sync_copy(fcb_hbm, b_v.at[pl.ds(0, 1)])

    for j in range(NCH):
      pltpu.async_copy(ut_hbm.at[uidx_v.at[j]],
                       urows_v.at[pl.ds(j * CHUNK, CHUNK)], usem)
      pltpu.async_copy(pt_hbm.at[pidx_v.at[j]],
                       prows_v.at[pl.ds(j * CHUNK, CHUNK)], psem)
    for j in range(NCH):
      pltpu.make_async_copy(ut_hbm.at[uidx_v.at[j]],
                            urows_v.at[pl.ds(j * CHUNK, CHUNK)], usem).wait()
      pltpu.make_async_copy(pt_hbm.at[pidx_v.at[j]],
                            prows_v.at[pl.ds(j * CHUNK, CHUNK)], psem).wait()

    wu = [w_v[pl.ds(j * L, L)] for j in range(E // L)]
    wp = [w_v[pl.ds(E + j * L, L)] for j in range(E // L)]
    b = b_v[...][0]

    lanes = lax.iota(jnp.int32, L)
    dnums = lax.GatherDimensionNumbers(
        offset_dims=(), collapsed_slice_dims=(0,), start_index_map=(0,))

    def _perm(v, idx):
      return lax.gather(v, idx[:, None], dnums, (1,),
                        mode=lax.GatherScatterMode.PROMISE_IN_BOUNDS)

    bfly = [lanes ^ (1 << k) for k in range(4)]

    @plsc.parallel_loop(0, BW // L, 1, unroll=2)
    def _grp(g):
      rbase = g * L
      out = jnp.zeros((L,), jnp.float32)
      for rr in range(L):
        r = rbase + rr
        acc = urows_v[r, pl.ds(0, L)] * wu[0]
        for j in range(1, E // L):
          acc += urows_v[r, pl.ds(j * L, L)] * wu[j]
        for j in range(E // L):
          acc += prows_v[r, pl.ds(j * L, L)] * wp[j]
        for k in range(4):
          acc = acc + _perm(acc, bfly[k])
        out = jnp.where(lanes == rr, acc, out)
      out_v[pl.ds(rbase, L)] = out + b

    pltpu.sync_copy(out_v, out_hbm.at[pl.ds(base, BW)])

  return k


def kernel(user, power, user_table, power_table, fc_w, fc_b):
  return _build(user.shape[0])(user, power, user_table, power_table,
                               fc_w, fc_b)


# speedup vs baseline: 4.0894x; 4.0894x over previous
import functools

import jax
import jax.numpy as jnp
from jax import lax
from jax.experimental import pallas as pl
from jax.experimental.pallas import tpu as pltpu
from jax.experimental.pallas import tpu_sc as plsc

L = 16
NC = 2
NS = 16
NW = NC * NS
E = 64
BLK = 4096


def _dense_body(w_ref, tu_ref, tp_ref, su_ref, sp_ref):
  w = w_ref[...]
  wu = w[0, :E].reshape(E, 1)
  wp = w[0, E:].reshape(E, 1)
  su_ref[...] = jnp.sum(tu_ref[...] * wu, axis=0)
  sp_ref[...] = jnp.sum(tp_ref[...] * wp, axis=0)


@functools.lru_cache(maxsize=None)
def _dense(n):
  grid = (n + BLK - 1) // BLK
  return pl.pallas_call(
      _dense_body,
      grid=(grid,),
      in_specs=[
          pl.BlockSpec((1, 2 * E), lambda i: (0, 0)),
          pl.BlockSpec((E, BLK), lambda i: (0, i)),
          pl.BlockSpec((E, BLK), lambda i: (0, i)),
      ],
      out_specs=[
          pl.BlockSpec((BLK,), lambda i: (i,)),
          pl.BlockSpec((BLK,), lambda i: (i,)),
      ],
      out_shape=[
          jax.ShapeDtypeStruct((n,), jnp.float32),
          jax.ShapeDtypeStruct((n,), jnp.float32),
      ],
  )


@functools.lru_cache(maxsize=None)
def _gather(B):
  BW = B // NW
  NCH = BW // 128
  mesh = plsc.VectorSubcoreMesh(core_axis_name="c", subcore_axis_name="s")

  @functools.partial(
      pl.kernel,
      out_type=jax.ShapeDtypeStruct((B,), jnp.float32),
      mesh=mesh,
      compiler_params=pltpu.CompilerParams(use_tc_tiling_on_sc=False),
      scratch_types=[
          pltpu.VMEM((BW,), jnp.int32),
          pltpu.VMEM((BW,), jnp.int32),
          pltpu.VMEM((NCH, 128), jnp.int32),
          pltpu.VMEM((NCH, 128), jnp.int32),
          pltpu.VMEM((BW, L), jnp.float32),
          pltpu.VMEM((BW, L), jnp.float32),
          pltpu.VMEM((L,), jnp.float32),
          pltpu.VMEM((BW,), jnp.float32),
          pltpu.SemaphoreType.DMA,
          pltpu.SemaphoreType.DMA,
      ],
  )
  def k(user_hbm, power_hbm, su_hbm, sp_hbm, fcb_hbm, out_hbm,
        uidx_v, pidx_v, uq_v, pq_v, ubuf_v, pbuf_v, b_v, out_v, usem, psem):
    wid = lax.axis_index("s") * NC + lax.axis_index("c")
    base = wid * BW

    pltpu.sync_copy(user_hbm.at[pl.ds(base, BW)], uidx_v)
    pltpu.sync_copy(power_hbm.at[pl.ds(base, BW)], pidx_v)
    pltpu.sync_copy(fcb_hbm, b_v.at[pl.ds(0, 1)])

    @plsc.parallel_loop(0, BW // L, 1, unroll=4)
    def _mkq(g):
      off = g * L
      uq_v[off // 128, pl.ds(off % 128, L)] = (
          lax.shift_right_logical(uidx_v[pl.ds(off, L)], 4))
      pq_v[off // 128, pl.ds(off % 128, L)] = (
          lax.shift_right_logical(pidx_v[pl.ds(off, L)], 4))

    for j in range(NCH):
      pltpu.async_copy(su_hbm.at[uq_v.at[j]],
                       ubuf_v.at[pl.ds(j * 128, 128)], usem)
      pltpu.async_copy(sp_hbm.at[pq_v.at[j]],
                       pbuf_v.at[pl.ds(j * 128, 128)], psem)
    for j in range(NCH):
      pltpu.make_async_copy(su_hbm.at[uq_v.at[j]],
                            ubuf_v.at[pl.ds(j * 128, 128)], usem).wait()
      pltpu.make_async_copy(sp_hbm.at[pq_v.at[j]],
                            pbuf_v.at[pl.ds(j * 128, 128)], psem).wait()

    lanes = lax.iota(jnp.int32, L)
    dnums = lax.GatherDimensionNumbers(
        offset_dims=(), collapsed_slice_dims=(0,), start_index_map=(0,))

    def _pick(v, m):
      idx = jnp.broadcast_to(m, (L,)).astype(jnp.int32)
      return lax.gather(v, idx[:, None], dnums, (1,),
                        mode=lax.GatherScatterMode.PROMISE_IN_BOUNDS)

    b = b_v[...][0]

    @plsc.parallel_loop(0, BW // L, 1, unroll=2)
    def _grp(g):
      off = g * L
      um = uidx_v[pl.ds(off, L)] & (L - 1)
      pm = pidx_v[pl.ds(off, L)] & (L - 1)
      out = jnp.zeros((L,), jnp.float32)
      for j in range(L):
        uv = ubuf_v[off + j, :]
        pv = pbuf_v[off + j, :]
        s = _pick(uv, um[j]) + _pick(pv, pm[j])
        out = jnp.where(lanes == j, s, out)
      out_v[pl.ds(off, L)] = out + b

    pltpu.sync_copy(out_v, out_hbm.at[pl.ds(base, BW)])

  return k


def kernel(user, power, user_table, power_table, fc_w, fc_b):
  n = user_table.shape[0]
  su, sp = _dense(n)(fc_w, user_table.T, power_table.T)
  nl = n // L
  out = _gather(user.shape[0])(user, power, su.reshape(nl, L),
                               sp.reshape(nl, L), fc_b)
  return out
